# baseline (speedup 1.0000x reference)
import functools

import jax
import jax.numpy as jnp
from jax import lax
from jax.experimental import pallas as pl
from jax.experimental.pallas import tpu as pltpu
from jax.experimental.pallas import tpu_sc as plsc

N = 10000
E = 320000
D = 128
R = 4

NC = 2
NS = 16
NW = NC * NS
NQ = 4
QL = D // NQ

RN = R * N
RNP = ((RN // NS + 7) // 8 * 8) * NS
STRIPE = RNP // NS
EPS = E // NS
BATCH = 128
EPSP = ((EPS + 511) // 512) * 512
NB = EPSP // BATCH
CHB = 1
NCH = NB // CHB
G16 = EPS // 16
TAIL_G = (EPSP - EPS) // 16

EPW = E // NW
EPWP = ((EPW + BATCH - 1) // BATCH) * BATCH
NB_CNT = EPWP // BATCH
G16_CNT = EPW // 16
TAIL_GC = (EPWP - EPW) // 16

_CHUNKS = [(o, min(128, STRIPE - o)) for o in range(0, STRIPE, 128)]

_MESH = plsc.VectorSubcoreMesh(core_axis_name="c", subcore_axis_name="s")
_SC_PARAMS = pltpu.CompilerParams(use_tc_tiling_on_sc=False)


def _fill_f32(ref, rows, cols, value):
    v16 = jnp.full((16,), value, jnp.float32)

    def body(t, _):
        j = t // (cols // 16)
        k = t % (cols // 16)
        ref[j, pl.ds(k * 16, 16)] = v16
        return _

    lax.fori_loop(0, rows * (cols // 16), body, None)


@functools.partial(
    pl.kernel,
    out_type=jax.ShapeDtypeStruct((NC, 2, RNP, QL), jnp.float32),
    mesh=_MESH,
    scratch_types=[
        pltpu.VMEM((EPSP,), jnp.int32),
        pltpu.VMEM((EPSP,), jnp.int32),
        pltpu.VMEM((2, CHB, BATCH, QL), jnp.float32),
        pltpu.VMEM_SHARED((RNP, QL), jnp.float32),
        pltpu.SemaphoreType.DMA,
        pltpu.SemaphoreType.DMA,
    ],
    compiler_params=_SC_PARAMS,
)
def _sc_edge_pass(hq, esrc, edst, et, acc, abuf, bbuf, rows, acc_sp,
                  sem0, sem1):
    c = lax.axis_index("c")
    s = lax.axis_index("s")

    base_e = s * EPS

    pltpu.sync_copy(edst.at[pl.ds(base_e, EPS)], abuf.at[pl.ds(0, EPS)])
    pltpu.sync_copy(et.at[pl.ds(base_e, EPS)], bbuf.at[pl.ds(0, EPS)])

    def mk_ridx(g, _):
        o = g * 16
        bbuf[pl.ds(o, 16)] = bbuf[pl.ds(o, 16)] * N + abuf[pl.ds(o, 16)]
        return _

    lax.fori_loop(0, G16, mk_ridx, None)
    pltpu.sync_copy(esrc.at[pl.ds(base_e, EPS)], abuf.at[pl.ds(0, EPS)])

    def mk_sidx(g, _):
        o = g * 16
        abuf[pl.ds(o, 16)] = abuf[pl.ds(o, 16)] * NQ + 2 * c
        return _

    lax.fori_loop(0, G16, mk_sidx, None)

    z16 = jnp.zeros((16,), jnp.int32)
    t16 = jnp.full((16,), RN, jnp.int32)

    def mk_tail(g, _):
        o = EPS + g * 16
        abuf[pl.ds(o, 16)] = z16
        bbuf[pl.ds(o, 16)] = t16
        return _

    lax.fori_loop(0, TAIL_G, mk_tail, None)

    def fire(ch, half, sem):
        for j in range(CHB):
            pltpu.async_copy(
                hq.at[abuf.at[pl.ds(ch * (CHB * BATCH) + j * BATCH, BATCH)]],
                rows.at[half, j], sem)

    def drain(half, sem):
        for j in range(CHB):
            pltpu.make_async_copy(
                hq.at[abuf.at[pl.ds(0, BATCH)]], rows.at[half, j], sem).wait()

    def scatter(ch, half):
        for j in range(CHB):
            pltpu.sync_copy(
                rows.at[half, j],
                acc_sp.at[bbuf.at[pl.ds(ch * (CHB * BATCH) + j * BATCH,
                                        BATCH)]],
                add=True)

    for p in range(2):
        if p == 1:
            def bump(g, _):
                o = g * 16
                abuf[pl.ds(o, 16)] = abuf[pl.ds(o, 16)] + 1
                return _

            lax.fori_loop(0, G16, bump, None)

        _fill_f32(rows.at[0, 0], 128, QL, 0.0)
        for (o, ln) in _CHUNKS:
            pltpu.sync_copy(rows.at[0, 0, pl.ds(0, ln)],
                            acc_sp.at[pl.ds(s * STRIPE + o, ln)])
        plsc.subcore_barrier()

        fire(0, 0, sem0)

        def pair_body(i, _):
            ch0 = 2 * i
            fire(ch0 + 1, 1, sem1)
            drain(0, sem0)
            scatter(ch0, 0)

            @pl.when(ch0 + 2 < NCH)
            def _fire_next():
                fire(ch0 + 2, 0, sem0)

            drain(1, sem1)
            scatter(ch0 + 1, 1)
            return _

        lax.fori_loop(0, NCH // 2, pair_body, None)
        plsc.subcore_barrier()

        for (o, ln) in _CHUNKS:
            pltpu.sync_copy(acc_sp.at[pl.ds(s * STRIPE + o, ln)],
                            acc.at[c, p, pl.ds(s * STRIPE + o, ln)])
        plsc.subcore_barrier()


@functools.partial(
    pl.kernel,
    out_type=jax.ShapeDtypeStruct((NC, RNP, 16), jnp.float32),
    mesh=_MESH,
    scratch_types=[
        pltpu.VMEM((EPWP,), jnp.int32),
        pltpu.VMEM((EPWP,), jnp.int32),
        pltpu.VMEM((BATCH, 16), jnp.float32),
        pltpu.VMEM((128, 16), jnp.float32),
        pltpu.VMEM_SHARED((RNP, 16), jnp.float32),
    ],
    compiler_params=_SC_PARAMS,
)
def _sc_count(edst, et, cnt, dstage, rbuf, ones, zbuf, cnt_sp):
    c = lax.axis_index("c")
    s = lax.axis_index("s")

    _fill_f32(ones, BATCH, 16, 1.0)
    _fill_f32(zbuf, 128, 16, 0.0)

    for (o, ln) in _CHUNKS:
        pltpu.sync_copy(zbuf.at[pl.ds(0, ln)],
                        cnt_sp.at[pl.ds(s * STRIPE + o, ln)])
    plsc.subcore_barrier()

    base_e = (s * NC + c) * EPW
    pltpu.sync_copy(edst.at[pl.ds(base_e, EPW)], dstage.at[pl.ds(0, EPW)])
    pltpu.sync_copy(et.at[pl.ds(base_e, EPW)], rbuf.at[pl.ds(0, EPW)])

    def mk_ridx(g, _):
        o = g * 16
        rbuf[pl.ds(o, 16)] = rbuf[pl.ds(o, 16)] * N + dstage[pl.ds(o, 16)]
        return _

    lax.fori_loop(0, G16_CNT, mk_ridx, None)

    t16 = jnp.full((16,), RN, jnp.int32)

    def mk_tail(g, _):
        rbuf[pl.ds(EPW + g * 16, 16)] = t16
        return _

    lax.fori_loop(0, TAIL_GC, mk_tail, None)

    def batch_body(t, _):
        pltpu.sync_copy(ones, cnt_sp.at[rbuf.at[pl.ds(t * BATCH, BATCH)]],
                        add=True)
        return _

    lax.fori_loop(0, NB_CNT, batch_body, None)
    plsc.subcore_barrier()

    for (o, ln) in _CHUNKS:
        pltpu.sync_copy(cnt_sp.at[pl.ds(s * STRIPE + o, ln)],
                        cnt.at[c, pl.ds(s * STRIPE + o, ln)])
    plsc.subcore_barrier()


BLK = 2000
_HIGH = lax.Precision.HIGHEST


def _layer_norm_in_kernel(h, g, b):
    mu = jnp.mean(h, axis=-1, keepdims=True)
    var = jnp.mean((h - mu) ** 2, axis=-1, keepdims=True)
    return (h - mu) * lax.rsqrt(var + 1e-5) * g + b


def _conv_block(x, accq, c0, c1, wrel, wroot, b, g, beta):
    o = jnp.dot(x, wroot, precision=_HIGH) + b
    scale = 1.0 / jnp.maximum(c0 + c1, 1.0)
    for r in range(R):
        ar = jnp.concatenate([accq[q, r] for q in range(NQ)], axis=-1)
        o = o + jnp.dot(ar * scale[:, r][:, None], wrel[r], precision=_HIGH)
    return _layer_norm_in_kernel(jax.nn.relu(o), g, beta)


def _tc_layer0_body(x_ref, acc_ref, c0_ref, c1_ref, wrel_ref, wroot_ref,
                    b_ref, g_ref, beta_ref, out_ref):
    out_ref[:] = _conv_block(x_ref[:], acc_ref[:], c0_ref[:], c1_ref[:],
                             wrel_ref[:], wroot_ref[:], b_ref[:], g_ref[:],
                             beta_ref[:])


def _tc_layer1_body(x_ref, acc_ref, c0_ref, c1_ref, wrel_ref, wroot_ref,
                    b_ref, g_ref, beta_ref, wd0_ref, bd0_ref, gd_ref,
                    betad_ref, wd1_ref, bd1_ref,
                    emb_ref, logit_ref, pred_ref):
    emb = _conv_block(x_ref[:], acc_ref[:], c0_ref[:], c1_ref[:],
                      wrel_ref[:], wroot_ref[:], b_ref[:], g_ref[:],
                      beta_ref[:])
    emb_ref[:] = emb
    z = _layer_norm_in_kernel(
        jax.nn.relu(jnp.dot(emb, wd0_ref[:], precision=_HIGH) + bd0_ref[:]),
        gd_ref[:], betad_ref[:])
    logits = jnp.dot(z, wd1_ref[:], precision=_HIGH) + bd1_ref[:]
    logit_ref[:] = logits
    pred_ref[:] = jax.nn.relu(logits)


def _row_spec():
    return pl.BlockSpec((BLK, D), lambda i: (i, 0))


def _common_specs():
    return [
        _row_spec(),
        pl.BlockSpec((NQ, R, BLK, QL), lambda i: (0, 0, i, 0)),
        pl.BlockSpec((BLK, R), lambda i: (i, 0)),
        pl.BlockSpec((BLK, R), lambda i: (i, 0)),
        pl.BlockSpec((R, D, D), lambda i: (0, 0, 0)),
        pl.BlockSpec((D, D), lambda i: (0, 0)),
        pl.BlockSpec((1, D), lambda i: (0, 0)),
        pl.BlockSpec((1, D), lambda i: (0, 0)),
        pl.BlockSpec((1, D), lambda i: (0, 0)),
    ]


def _tc_layer0(x, acc, c0, c1, wrel, wroot, b, g, beta):
    return pl.pallas_call(
        _tc_layer0_body,
        grid=(N // BLK,),
        in_specs=_common_specs(),
        out_specs=_row_spec(),
        out_shape=jax.ShapeDtypeStruct((N, D), jnp.float32),
    )(x, acc, c0, c1, wrel, wroot, b, g, beta)


def _tc_layer1(x, acc, c0, c1, wrel, wroot, b, g, beta,
               wd0, bd0, gd, betad, wd1, bd1):
    specs = _common_specs() + [
        pl.BlockSpec((D, D), lambda i: (0, 0)),
        pl.BlockSpec((1, D), lambda i: (0, 0)),
        pl.BlockSpec((1, D), lambda i: (0, 0)),
        pl.BlockSpec((1, D), lambda i: (0, 0)),
        pl.BlockSpec((D, D), lambda i: (0, 0)),
        pl.BlockSpec((1, D), lambda i: (0, 0)),
    ]
    return pl.pallas_call(
        _tc_layer1_body,
        grid=(N // BLK,),
        in_specs=specs,
        out_specs=(_row_spec(), _row_spec(), _row_spec()),
        out_shape=(
            jax.ShapeDtypeStruct((N, D), jnp.float32),
            jax.ShapeDtypeStruct((N, D), jnp.float32),
            jax.ShapeDtypeStruct((N, D), jnp.float32),
        ),
    )(x, acc, c0, c1, wrel, wroot, b, g, beta, wd0, bd0, gd, betad, wd1, bd1)


def kernel(x, edge_index, edge_type, W_rel0, W_root0, b0, g0, beta0,
           W_rel1, W_root1, b1, g1, beta1, Wd0, bd0, gd, betad, Wd1, bd1):
    esrc = edge_index[0].astype(jnp.int32)
    edst = edge_index[1].astype(jnp.int32)
    et = edge_type.astype(jnp.int32)

    cnth = _sc_count(edst, et)
    c0 = cnth[0, :RN, 0].reshape(R, N).T
    c1 = cnth[1, :RN, 0].reshape(R, N).T

    acc0 = _sc_edge_pass(x.reshape(N * NQ, QL), esrc, edst, et)

    b0r, g0r, beta0r = b0[None, :], g0[None, :], beta0[None, :]
    b1r, g1r, beta1r = b1[None, :], g1[None, :], beta1[None, :]
    bd0r, gdr, betadr = bd0[None, :], gd[None, :], betad[None, :]
    wd1p = jnp.pad(Wd1, ((0, 0), (0, D - Wd1.shape[1])))
    bd1p = jnp.pad(bd1[None, :], ((0, 0), (0, D - bd1.shape[0])))

    h1 = _tc_layer0(x, acc0[:, :, :RN].reshape(NQ, R, N, QL), c0, c1,
                    W_rel0, W_root0, b0r, g0r, beta0r)
    acc1 = _sc_edge_pass(h1.reshape(N * NQ, QL), esrc, edst, et)
    emb, logits_p, preds_p = _tc_layer1(
        h1, acc1[:, :, :RN].reshape(NQ, R, N, QL), c0, c1,
        W_rel1, W_root1, b1r, g1r, beta1r,
        Wd0, bd0r, gdr, betadr, wd1p, bd1p)
    return preds_p[:, :1], emb, logits_p[:, :1]


# speedup vs baseline: 1.7587x; 1.7587x over previous
import functools

import jax
import jax.numpy as jnp
from jax import lax
from jax.experimental import pallas as pl
from jax.experimental.pallas import tpu as pltpu
from jax.experimental.pallas import tpu_sc as plsc

N = 10000
E = 320000
D = 128
R = 4

NC = 2
NS = 16
NW = NC * NS
NQ = 4
QL = D // NQ

RN = R * N
RNP = ((RN // NS + 7) // 8 * 8) * NS
STRIPE = RNP // NS
EPS = E // NS
BATCH = 128
SEGP = 5120
CHB = 4
NCH = SEGP // (CHB * BATCH)
_SEGS = [(o, min(SEGP, EPS - o)) for o in range(0, EPS, SEGP)]

EPW = E // NW
EPWP = ((EPW + BATCH - 1) // BATCH) * BATCH
NB_CNT = EPWP // BATCH
G16_CNT = EPW // 16
TAIL_GC = (EPWP - EPW) // 16

_CHUNKS = [(o, min(128, STRIPE - o)) for o in range(0, STRIPE, 128)]

_MESH = plsc.VectorSubcoreMesh(core_axis_name="c", subcore_axis_name="s")
_SC_PARAMS = pltpu.CompilerParams(use_tc_tiling_on_sc=False)


def _fill_f32(ref, rows, cols, value):
    v16 = jnp.full((16,), value, jnp.float32)

    def body(t, _):
        j = t // (cols // 16)
        k = t % (cols // 16)
        ref[j, pl.ds(k * 16, 16)] = v16
        return _

    lax.fori_loop(0, rows * (cols // 16), body, None)


@functools.partial(
    pl.kernel,
    out_type=jax.ShapeDtypeStruct((NC, 2, RNP, QL), jnp.float32),
    mesh=_MESH,
    scratch_types=[
        pltpu.VMEM((SEGP,), jnp.int32),
        pltpu.VMEM((SEGP,), jnp.int32),
        pltpu.VMEM((2, CHB, BATCH, QL), jnp.float32),
        pltpu.VMEM_SHARED((RNP, QL), jnp.float32),
        pltpu.SemaphoreType.DMA,
        pltpu.SemaphoreType.DMA,
    ],
    compiler_params=_SC_PARAMS,
)
def _sc_edge_pass(hq, esrc, edst, et, acc, abuf, bbuf, rows, acc_sp,
                  sem0, sem1):
    c = lax.axis_index("c")
    s = lax.axis_index("s")

    base_e = s * EPS

    def fire(ch, half, sem):
        for j in range(CHB):
            pltpu.async_copy(
                hq.at[abuf.at[pl.ds(ch * (CHB * BATCH) + j * BATCH, BATCH)]],
                rows.at[half, j], sem)

    def drain(half, sem):
        for j in range(CHB):
            pltpu.make_async_copy(
                hq.at[abuf.at[pl.ds(0, BATCH)]], rows.at[half, j], sem).wait()

    def scatter(ch, half):
        for j in range(CHB):
            pltpu.sync_copy(
                rows.at[half, j],
                acc_sp.at[bbuf.at[pl.ds(ch * (CHB * BATCH) + j * BATCH,
                                        BATCH)]],
                add=True)

    t16 = jnp.full((16,), RN, jnp.int32)

    for p in range(2):
        qv = 2 * c + p

        _fill_f32(rows.at[0, 0], 128, QL, 0.0)
        for (o, ln) in _CHUNKS:
            pltpu.sync_copy(rows.at[0, 0, pl.ds(0, ln)],
                            acc_sp.at[pl.ds(s * STRIPE + o, ln)])
        plsc.subcore_barrier()

        for (so, sn) in _SEGS:
            pltpu.sync_copy(edst.at[pl.ds(base_e + so, sn)],
                            abuf.at[pl.ds(0, sn)])
            pltpu.sync_copy(et.at[pl.ds(base_e + so, sn)],
                            bbuf.at[pl.ds(0, sn)])

            def mk_ridx(g, _):
                o = g * 16
                bbuf[pl.ds(o, 16)] = (bbuf[pl.ds(o, 16)] * N
                                      + abuf[pl.ds(o, 16)])
                return _

            lax.fori_loop(0, sn // 16, mk_ridx, None)
            pltpu.sync_copy(esrc.at[pl.ds(base_e + so, sn)],
                            abuf.at[pl.ds(0, sn)])

            def mk_sidx(g, _, qv=qv):
                o = g * 16
                abuf[pl.ds(o, 16)] = abuf[pl.ds(o, 16)] * NQ + qv
                return _

            lax.fori_loop(0, sn // 16, mk_sidx, None)

            if sn < SEGP:
                def mk_tail(g, _):
                    bbuf[pl.ds(sn + g * 16, 16)] = t16
                    return _

                lax.fori_loop(0, (SEGP - sn) // 16, mk_tail, None)

            fire(0, 0, sem0)

            def pair_body(i, _):
                ch0 = 2 * i
                fire(ch0 + 1, 1, sem1)
                drain(0, sem0)
                scatter(ch0, 0)

                @pl.when(ch0 + 2 < NCH)
                def _fire_next():
                    fire(ch0 + 2, 0, sem0)

                drain(1, sem1)
                scatter(ch0 + 1, 1)
                return _

            lax.fori_loop(0, NCH // 2, pair_body, None)

        plsc.subcore_barrier()

        for (o, ln) in _CHUNKS:
            pltpu.sync_copy(acc_sp.at[pl.ds(s * STRIPE + o, ln)],
                            acc.at[c, p, pl.ds(s * STRIPE + o, ln)])
        plsc.subcore_barrier()


@functools.partial(
    pl.kernel,
    out_type=jax.ShapeDtypeStruct((NC, RNP, 16), jnp.float32),
    mesh=_MESH,
    scratch_types=[
        pltpu.VMEM((EPWP,), jnp.int32),
        pltpu.VMEM((EPWP,), jnp.int32),
        pltpu.VMEM((BATCH, 16), jnp.float32),
        pltpu.VMEM((128, 16), jnp.float32),
        pltpu.VMEM_SHARED((RNP, 16), jnp.float32),
    ],
    compiler_params=_SC_PARAMS,
)
def _sc_count(edst, et, cnt, dstage, rbuf, ones, zbuf, cnt_sp):
    c = lax.axis_index("c")
    s = lax.axis_index("s")

    _fill_f32(ones, BATCH, 16, 1.0)
    _fill_f32(zbuf, 128, 16, 0.0)

    for (o, ln) in _CHUNKS:
        pltpu.sync_copy(zbuf.at[pl.ds(0, ln)],
                        cnt_sp.at[pl.ds(s * STRIPE + o, ln)])
    plsc.subcore_barrier()

    base_e = (s * NC + c) * EPW
    pltpu.sync_copy(edst.at[pl.ds(base_e, EPW)], dstage.at[pl.ds(0, EPW)])
    pltpu.sync_copy(et.at[pl.ds(base_e, EPW)], rbuf.at[pl.ds(0, EPW)])

    def mk_ridx(g, _):
        o = g * 16
        rbuf[pl.ds(o, 16)] = rbuf[pl.ds(o, 16)] * N + dstage[pl.ds(o, 16)]
        return _

    lax.fori_loop(0, G16_CNT, mk_ridx, None)

    t16 = jnp.full((16,), RN, jnp.int32)

    def mk_tail(g, _):
        rbuf[pl.ds(EPW + g * 16, 16)] = t16
        return _

    lax.fori_loop(0, TAIL_GC, mk_tail, None)

    def batch_body(t, _):
        pltpu.sync_copy(ones, cnt_sp.at[rbuf.at[pl.ds(t * BATCH, BATCH)]],
                        add=True)
        return _

    lax.fori_loop(0, NB_CNT, batch_body, None)
    plsc.subcore_barrier()

    for (o, ln) in _CHUNKS:
        pltpu.sync_copy(cnt_sp.at[pl.ds(s * STRIPE + o, ln)],
                        cnt.at[c, pl.ds(s * STRIPE + o, ln)])
    plsc.subcore_barrier()


BLK = 2000
_HIGH = lax.Precision.HIGHEST


def _layer_norm_in_kernel(h, g, b):
    mu = jnp.mean(h, axis=-1, keepdims=True)
    var = jnp.mean((h - mu) ** 2, axis=-1, keepdims=True)
    return (h - mu) * lax.rsqrt(var + 1e-5) * g + b


def _conv_block(x, accq, c0, c1, wrel, wroot, b, g, beta):
    o = jnp.dot(x, wroot, precision=_HIGH) + b
    scale = 1.0 / jnp.maximum(c0 + c1, 1.0)
    for r in range(R):
        ar = jnp.concatenate([accq[q, r] for q in range(NQ)], axis=-1)
        o = o + jnp.dot(ar * scale[:, r][:, None], wrel[r], precision=_HIGH)
    return _layer_norm_in_kernel(jax.nn.relu(o), g, beta)


def _tc_layer0_body(x_ref, acc_ref, c0_ref, c1_ref, wrel_ref, wroot_ref,
                    b_ref, g_ref, beta_ref, out_ref):
    out_ref[:] = _conv_block(x_ref[:], acc_ref[:], c0_ref[:], c1_ref[:],
                             wrel_ref[:], wroot_ref[:], b_ref[:], g_ref[:],
                             beta_ref[:])


def _tc_layer1_body(x_ref, acc_ref, c0_ref, c1_ref, wrel_ref, wroot_ref,
                    b_ref, g_ref, beta_ref, wd0_ref, bd0_ref, gd_ref,
                    betad_ref, wd1_ref, bd1_ref,
                    emb_ref, logit_ref, pred_ref):
    emb = _conv_block(x_ref[:], acc_ref[:], c0_ref[:], c1_ref[:],
                      wrel_ref[:], wroot_ref[:], b_ref[:], g_ref[:],
                      beta_ref[:])
    emb_ref[:] = emb
    z = _layer_norm_in_kernel(
        jax.nn.relu(jnp.dot(emb, wd0_ref[:], precision=_HIGH) + bd0_ref[:]),
        gd_ref[:], betad_ref[:])
    logits = jnp.dot(z, wd1_ref[:], precision=_HIGH) + bd1_ref[:]
    logit_ref[:] = logits
    pred_ref[:] = jax.nn.relu(logits)


def _row_spec():
    return pl.BlockSpec((BLK, D), lambda i: (i, 0))


def _common_specs():
    return [
        _row_spec(),
        pl.BlockSpec((NQ, R, BLK, QL), lambda i: (0, 0, i, 0)),
        pl.BlockSpec((BLK, R), lambda i: (i, 0)),
        pl.BlockSpec((BLK, R), lambda i: (i, 0)),
        pl.BlockSpec((R, D, D), lambda i: (0, 0, 0)),
        pl.BlockSpec((D, D), lambda i: (0, 0)),
        pl.BlockSpec((1, D), lambda i: (0, 0)),
        pl.BlockSpec((1, D), lambda i: (0, 0)),
        pl.BlockSpec((1, D), lambda i: (0, 0)),
    ]


def _tc_layer0(x, acc, c0, c1, wrel, wroot, b, g, beta):
    return pl.pallas_call(
        _tc_layer0_body,
        grid=(N // BLK,),
        in_specs=_common_specs(),
        out_specs=_row_spec(),
        out_shape=jax.ShapeDtypeStruct((N, D), jnp.float32),
    )(x, acc, c0, c1, wrel, wroot, b, g, beta)


def _tc_layer1(x, acc, c0, c1, wrel, wroot, b, g, beta,
               wd0, bd0, gd, betad, wd1, bd1):
    specs = _common_specs() + [
        pl.BlockSpec((D, D), lambda i: (0, 0)),
        pl.BlockSpec((1, D), lambda i: (0, 0)),
        pl.BlockSpec((1, D), lambda i: (0, 0)),
        pl.BlockSpec((1, D), lambda i: (0, 0)),
        pl.BlockSpec((D, D), lambda i: (0, 0)),
        pl.BlockSpec((1, D), lambda i: (0, 0)),
    ]
    return pl.pallas_call(
        _tc_layer1_body,
        grid=(N // BLK,),
        in_specs=specs,
        out_specs=(_row_spec(), _row_spec(), _row_spec()),
        out_shape=(
            jax.ShapeDtypeStruct((N, D), jnp.float32),
            jax.ShapeDtypeStruct((N, D), jnp.float32),
            jax.ShapeDtypeStruct((N, D), jnp.float32),
        ),
    )(x, acc, c0, c1, wrel, wroot, b, g, beta, wd0, bd0, gd, betad, wd1, bd1)


def kernel(x, edge_index, edge_type, W_rel0, W_root0, b0, g0, beta0,
           W_rel1, W_root1, b1, g1, beta1, Wd0, bd0, gd, betad, Wd1, bd1):
    esrc = edge_index[0].astype(jnp.int32)
    edst = edge_index[1].astype(jnp.int32)
    et = edge_type.astype(jnp.int32)

    cnth = _sc_count(edst, et)
    c0 = cnth[0, :RN, 0].reshape(R, N).T
    c1 = cnth[1, :RN, 0].reshape(R, N).T

    acc0 = _sc_edge_pass(x.reshape(N * NQ, QL), esrc, edst, et)

    b0r, g0r, beta0r = b0[None, :], g0[None, :], beta0[None, :]
    b1r, g1r, beta1r = b1[None, :], g1[None, :], beta1[None, :]
    bd0r, gdr, betadr = bd0[None, :], gd[None, :], betad[None, :]
    wd1p = jnp.pad(Wd1, ((0, 0), (0, D - Wd1.shape[1])))
    bd1p = jnp.pad(bd1[None, :], ((0, 0), (0, D - bd1.shape[0])))

    h1 = _tc_layer0(x, acc0[:, :, :RN].reshape(NQ, R, N, QL), c0, c1,
                    W_rel0, W_root0, b0r, g0r, beta0r)
    acc1 = _sc_edge_pass(h1.reshape(N * NQ, QL), esrc, edst, et)
    emb, logits_p, preds_p = _tc_layer1(
        h1, acc1[:, :, :RN].reshape(NQ, R, N, QL), c0, c1,
        W_rel1, W_root1, b1r, g1r, beta1r,
        Wd0, bd0r, gdr, betadr, wd1p, bd1p)
    return preds_p[:, :1], emb, logits_p[:, :1]


# speedup vs baseline: 2.0061x; 1.1406x over previous
import functools

import jax
import jax.numpy as jnp
from jax import lax
from jax.experimental import pallas as pl
from jax.experimental.pallas import tpu as pltpu
from jax.experimental.pallas import tpu_sc as plsc

N = 10000
E = 320000
D = 128
R = 4

NC = 2
NS = 16
NW = NC * NS
NQ = 4
QL = D // NQ

RN = R * N
RNP = ((RN // NS + 7) // 8 * 8) * NS
STRIPE = RNP // NS
EPS = E // NS
BATCH = 128
SEGP = 5120
CHB = 4
NCH = SEGP // (CHB * BATCH)
_SEGS = [(o, min(SEGP, EPS - o)) for o in range(0, EPS, SEGP)]

EPW = E // NW
EPWP = ((EPW + BATCH - 1) // BATCH) * BATCH
NB_CNT = EPWP // BATCH
G16_CNT = EPW // 16
TAIL_GC = (EPWP - EPW) // 16

_CHUNKS = [(o, min(128, STRIPE - o)) for o in range(0, STRIPE, 128)]

_MESH = plsc.VectorSubcoreMesh(core_axis_name="c", subcore_axis_name="s")
_SC_PARAMS = pltpu.CompilerParams(use_tc_tiling_on_sc=False)


def _fill_f32(ref, rows, cols, value):
    v16 = jnp.full((16,), value, jnp.float32)

    def body(t, _):
        j = t // (cols // 16)
        k = t % (cols // 16)
        ref[j, pl.ds(k * 16, 16)] = v16
        return _

    lax.fori_loop(0, rows * (cols // 16), body, None)


@functools.partial(
    pl.kernel,
    out_type=jax.ShapeDtypeStruct((NC, 2, RNP, QL), jnp.float32),
    mesh=_MESH,
    scratch_types=[
        pltpu.VMEM((SEGP,), jnp.int32),
        pltpu.VMEM((SEGP,), jnp.int32),
        pltpu.VMEM((2, CHB, BATCH, QL), jnp.float32),
        pltpu.VMEM_SHARED((RNP, QL), jnp.float32),
        pltpu.SemaphoreType.DMA,
        pltpu.SemaphoreType.DMA,
    ],
    compiler_params=_SC_PARAMS,
)
def _sc_edge_pass(hq, esrc, edst, et, acc, abuf, bbuf, rows, acc_sp,
                  sem0, sem1):
    c = lax.axis_index("c")
    s = lax.axis_index("s")

    base_e = s * EPS

    def fire(ch, half, sem):
        for j in range(CHB):
            pltpu.async_copy(
                hq.at[abuf.at[pl.ds(ch * (CHB * BATCH) + j * BATCH, BATCH)]],
                rows.at[half, j], sem)

    def drain(half, sem):
        for j in range(CHB):
            pltpu.make_async_copy(
                hq.at[abuf.at[pl.ds(0, BATCH)]], rows.at[half, j], sem).wait()

    def scatter(ch, half):
        for j in range(CHB):
            pltpu.sync_copy(
                rows.at[half, j],
                acc_sp.at[bbuf.at[pl.ds(ch * (CHB * BATCH) + j * BATCH,
                                        BATCH)]],
                add=True)

    t16 = jnp.full((16,), RN, jnp.int32)

    for p in range(2):
        qv = 2 * c + p

        _fill_f32(rows.at[0, 0], 128, QL, 0.0)
        for (o, ln) in _CHUNKS:
            pltpu.sync_copy(rows.at[0, 0, pl.ds(0, ln)],
                            acc_sp.at[pl.ds(s * STRIPE + o, ln)])
        plsc.subcore_barrier()

        for (so, sn) in _SEGS:
            pltpu.sync_copy(edst.at[pl.ds(base_e + so, sn)],
                            abuf.at[pl.ds(0, sn)])
            pltpu.sync_copy(et.at[pl.ds(base_e + so, sn)],
                            bbuf.at[pl.ds(0, sn)])

            def mk_ridx(g, _):
                o = g * 16
                bbuf[pl.ds(o, 16)] = (bbuf[pl.ds(o, 16)] * N
                                      + abuf[pl.ds(o, 16)])
                return _

            lax.fori_loop(0, sn // 16, mk_ridx, None)
            pltpu.sync_copy(esrc.at[pl.ds(base_e + so, sn)],
                            abuf.at[pl.ds(0, sn)])

            def mk_sidx(g, _, qv=qv):
                o = g * 16
                abuf[pl.ds(o, 16)] = abuf[pl.ds(o, 16)] * NQ + qv
                return _

            lax.fori_loop(0, sn // 16, mk_sidx, None)

            if sn < SEGP:
                def mk_tail(g, _):
                    bbuf[pl.ds(sn + g * 16, 16)] = t16
                    return _

                lax.fori_loop(0, (SEGP - sn) // 16, mk_tail, None)

            fire(0, 0, sem0)

            def pair_body(i, _):
                ch0 = 2 * i
                fire(ch0 + 1, 1, sem1)
                drain(0, sem0)
                scatter(ch0, 0)

                @pl.when(ch0 + 2 < NCH)
                def _fire_next():
                    fire(ch0 + 2, 0, sem0)

                drain(1, sem1)
                scatter(ch0 + 1, 1)
                return _

            lax.fori_loop(0, NCH // 2, pair_body, None)

        plsc.subcore_barrier()

        for (o, ln) in _CHUNKS:
            pltpu.sync_copy(acc_sp.at[pl.ds(s * STRIPE + o, ln)],
                            acc.at[c, p, pl.ds(s * STRIPE + o, ln)])
        plsc.subcore_barrier()


@functools.partial(
    pl.kernel,
    out_type=jax.ShapeDtypeStruct((NC, RNP, 16), jnp.float32),
    mesh=_MESH,
    scratch_types=[
        pltpu.VMEM((EPWP,), jnp.int32),
        pltpu.VMEM((EPWP,), jnp.int32),
        pltpu.VMEM((BATCH, 16), jnp.float32),
        pltpu.VMEM((128, 16), jnp.float32),
        pltpu.VMEM_SHARED((RNP, 16), jnp.float32),
    ],
    compiler_params=_SC_PARAMS,
)
def _sc_count(edst, et, cnt, dstage, rbuf, ones, zbuf, cnt_sp):
    c = lax.axis_index("c")
    s = lax.axis_index("s")

    _fill_f32(ones, BATCH, 16, 1.0)
    _fill_f32(zbuf, 128, 16, 0.0)

    for (o, ln) in _CHUNKS:
        pltpu.sync_copy(zbuf.at[pl.ds(0, ln)],
                        cnt_sp.at[pl.ds(s * STRIPE + o, ln)])
    plsc.subcore_barrier()

    base_e = (s * NC + c) * EPW
    pltpu.sync_copy(edst.at[pl.ds(base_e, EPW)], dstage.at[pl.ds(0, EPW)])
    pltpu.sync_copy(et.at[pl.ds(base_e, EPW)], rbuf.at[pl.ds(0, EPW)])

    def mk_ridx(g, _):
        o = g * 16
        rbuf[pl.ds(o, 16)] = rbuf[pl.ds(o, 16)] * N + dstage[pl.ds(o, 16)]
        return _

    lax.fori_loop(0, G16_CNT, mk_ridx, None)

    t16 = jnp.full((16,), RN, jnp.int32)

    def mk_tail(g, _):
        rbuf[pl.ds(EPW + g * 16, 16)] = t16
        return _

    lax.fori_loop(0, TAIL_GC, mk_tail, None)

    def batch_body(t, _):
        pltpu.sync_copy(ones, cnt_sp.at[rbuf.at[pl.ds(t * BATCH, BATCH)]],
                        add=True)
        return _

    lax.fori_loop(0, NB_CNT, batch_body, None)
    plsc.subcore_barrier()

    for (o, ln) in _CHUNKS:
        pltpu.sync_copy(cnt_sp.at[pl.ds(s * STRIPE + o, ln)],
                        cnt.at[c, pl.ds(s * STRIPE + o, ln)])
    plsc.subcore_barrier()


BLK = 2000
_HIGH = lax.Precision.HIGHEST


def _layer_norm_in_kernel(h, g, b):
    mu = jnp.mean(h, axis=-1, keepdims=True)
    var = jnp.mean((h - mu) ** 2, axis=-1, keepdims=True)
    return (h - mu) * lax.rsqrt(var + 1e-5) * g + b


def _conv_block(x, acc_refs, c0, c1, wrel, wroot, b, g, beta):
    o = jnp.dot(x, wroot, precision=_HIGH) + b
    scale = 1.0 / jnp.maximum(c0 + c1, 1.0)
    for r in range(R):
        ar = jnp.concatenate(
            [acc_refs[q * R + r][0, 0] for q in range(NQ)], axis=-1)
        o = o + jnp.dot(ar * scale[:, r][:, None], wrel[r], precision=_HIGH)
    return _layer_norm_in_kernel(jax.nn.relu(o), g, beta)


_NACC = NQ * R


def _tc_layer0_body(x_ref, *rest):
    acc_refs = rest[:_NACC]
    (c0_ref, c1_ref, wrel_ref, wroot_ref, b_ref, g_ref, beta_ref,
     out_ref) = rest[_NACC:]
    out_ref[:] = _conv_block(x_ref[:], acc_refs, c0_ref[:], c1_ref[:],
                             wrel_ref[:], wroot_ref[:], b_ref[:], g_ref[:],
                             beta_ref[:])


def _tc_layer1_body(x_ref, *rest):
    acc_refs = rest[:_NACC]
    (c0_ref, c1_ref, wrel_ref, wroot_ref, b_ref, g_ref, beta_ref,
     wd0_ref, bd0_ref, gd_ref, betad_ref, wd1_ref, bd1_ref,
     emb_ref, logit_ref, pred_ref) = rest[_NACC:]
    emb = _conv_block(x_ref[:], acc_refs, c0_ref[:], c1_ref[:],
                      wrel_ref[:], wroot_ref[:], b_ref[:], g_ref[:],
                      beta_ref[:])
    emb_ref[:] = emb
    z = _layer_norm_in_kernel(
        jax.nn.relu(jnp.dot(emb, wd0_ref[:], precision=_HIGH) + bd0_ref[:]),
        gd_ref[:], betad_ref[:])
    logits = jnp.dot(z, wd1_ref[:], precision=_HIGH) + bd1_ref[:]
    logit_ref[:] = logits
    pred_ref[:] = jax.nn.relu(logits)


def _row_spec():
    return pl.BlockSpec((BLK, D), lambda i: (i, 0))


def _acc_spec(q, r):
    return pl.BlockSpec(
        (1, 1, BLK, QL),
        lambda i, c=q // 2, p=q % 2, r=r: (c, p, r * (N // BLK) + i, 0))


def _common_specs():
    return ([_row_spec()]
            + [_acc_spec(q, r) for q in range(NQ) for r in range(R)]
            + [
        pl.BlockSpec((BLK, R), lambda i: (i, 0)),
        pl.BlockSpec((BLK, R), lambda i: (i, 0)),
        pl.BlockSpec((R, D, D), lambda i: (0, 0, 0)),
        pl.BlockSpec((D, D), lambda i: (0, 0)),
        pl.BlockSpec((1, D), lambda i: (0, 0)),
        pl.BlockSpec((1, D), lambda i: (0, 0)),
        pl.BlockSpec((1, D), lambda i: (0, 0)),
    ])


def _tc_layer0(x, acc, c0, c1, wrel, wroot, b, g, beta):
    return pl.pallas_call(
        _tc_layer0_body,
        grid=(N // BLK,),
        in_specs=_common_specs(),
        out_specs=_row_spec(),
        out_shape=jax.ShapeDtypeStruct((N, D), jnp.float32),
    )(x, *([acc] * _NACC), c0, c1, wrel, wroot, b, g, beta)


def _tc_layer1(x, acc, c0, c1, wrel, wroot, b, g, beta,
               wd0, bd0, gd, betad, wd1, bd1):
    specs = _common_specs() + [
        pl.BlockSpec((D, D), lambda i: (0, 0)),
        pl.BlockSpec((1, D), lambda i: (0, 0)),
        pl.BlockSpec((1, D), lambda i: (0, 0)),
        pl.BlockSpec((1, D), lambda i: (0, 0)),
        pl.BlockSpec((D, D), lambda i: (0, 0)),
        pl.BlockSpec((1, D), lambda i: (0, 0)),
    ]
    return pl.pallas_call(
        _tc_layer1_body,
        grid=(N // BLK,),
        in_specs=specs,
        out_specs=(_row_spec(), _row_spec(), _row_spec()),
        out_shape=(
            jax.ShapeDtypeStruct((N, D), jnp.float32),
            jax.ShapeDtypeStruct((N, D), jnp.float32),
            jax.ShapeDtypeStruct((N, D), jnp.float32),
        ),
    )(x, *([acc] * _NACC), c0, c1, wrel, wroot, b, g, beta,
      wd0, bd0, gd, betad, wd1, bd1)


def kernel(x, edge_index, edge_type, W_rel0, W_root0, b0, g0, beta0,
           W_rel1, W_root1, b1, g1, beta1, Wd0, bd0, gd, betad, Wd1, bd1):
    esrc = edge_index[0].astype(jnp.int32)
    edst = edge_index[1].astype(jnp.int32)
    et = edge_type.astype(jnp.int32)

    cnth = _sc_count(edst, et)
    c0 = cnth[0, :RN, 0].reshape(R, N).T
    c1 = cnth[1, :RN, 0].reshape(R, N).T

    acc0 = _sc_edge_pass(x.reshape(N * NQ, QL), esrc, edst, et)

    b0r, g0r, beta0r = b0[None, :], g0[None, :], beta0[None, :]
    b1r, g1r, beta1r = b1[None, :], g1[None, :], beta1[None, :]
    bd0r, gdr, betadr = bd0[None, :], gd[None, :], betad[None, :]
    wd1p = jnp.pad(Wd1, ((0, 0), (0, D - Wd1.shape[1])))
    bd1p = jnp.pad(bd1[None, :], ((0, 0), (0, D - bd1.shape[0])))

    h1 = _tc_layer0(x, acc0, c0, c1, W_rel0, W_root0, b0r, g0r, beta0r)
    acc1 = _sc_edge_pass(h1.reshape(N * NQ, QL), esrc, edst, et)
    emb, logits_p, preds_p = _tc_layer1(
        h1, acc1, c0, c1, W_rel1, W_root1, b1r, g1r, beta1r,
        Wd0, bd0r, gdr, betadr, wd1p, bd1p)
    return preds_p[:, :1], emb, logits_p[:, :1]
